# baseline (speedup 1.0000x reference)
import jax
import jax.numpy as jnp
from jax.experimental import pallas as pl
from jax.experimental.pallas import tpu as pltpu

B, S, H = 1, 2048, 1024
NH, NKV, HD = 16, 4, 64
E, TOPK, I = 8, 2, 768
EPS = 1e-06
THETA = 1000000.0
REP = NH // NKV

SB = 256
BQ = 256
BLK = 128
PN = S * TOPK + E * BLK
NB = PN // BLK

_HALF = HD // 2
_NEG = -1e30

f32 = jnp.float32
bf16 = jnp.bfloat16


def _rmsnorm(x, w):
    v = jnp.mean(jnp.square(x), axis=-1, keepdims=True)
    return x * jax.lax.rsqrt(v + EPS) * w


def _rope(x, pos):
    inv = 1.0 / (THETA ** (jnp.arange(0, HD, 2, dtype=f32) / HD))
    freqs = pos.astype(f32)[..., None] * inv
    cos = jnp.cos(freqs)[:, :, None, :]
    sin = jnp.sin(freqs)[:, :, None, :]
    x1, x2 = x[..., :_HALF], x[..., _HALF:]
    return jnp.concatenate([x1 * cos - x2 * sin, x2 * cos + x1 * sin], axis=-1)


def _qkv_kernel(xn_ref, wq_ref, wk_ref, wv_ref, q_ref, k_ref, v_ref):
    xn = xn_ref[...]
    q_ref[...] = jnp.dot(xn, wq_ref[...].astype(bf16), preferred_element_type=f32)
    k_ref[...] = jnp.dot(xn, wk_ref[...].astype(bf16), preferred_element_type=f32)
    v_ref[...] = jnp.dot(xn, wv_ref[...].astype(bf16),
                         preferred_element_type=f32).astype(bf16)


def _attn_kernel(q_ref, k_ref, v_ref, o_ref, r_ref):
    qb = pl.program_id(1)
    q = q_ref[0]
    s = jax.lax.dot_general(q, k_ref[0], (((1,), (1,)), ((), ())),
                            preferred_element_type=f32) / jnp.sqrt(float(HD))
    rid = qb * BQ + jax.lax.broadcasted_iota(jnp.int32, (BQ, S), 0)
    cid = jax.lax.broadcasted_iota(jnp.int32, (BQ, S), 1)
    s = jnp.where(rid >= cid, s, _NEG)
    m = jnp.max(s, axis=-1, keepdims=True)
    ex = jnp.exp(s - m)
    parts = []
    for j in range(S // 128):
        a = ex[:, j * 128:(j + 1) * 128]
        w = 128
        while w > 1:
            w //= 2
            a = a[:, :w] + a[:, w:2 * w]
        parts.append(a)
    while len(parts) > 1:
        hn = len(parts) // 2
        parts = [parts[i] + parts[i + hn] for i in range(hn)]
    l = parts[0]
    r_ref[...] = 1.0 / l
    p = (ex * r_ref[...]).astype(bf16)
    ot = jax.lax.dot_general(v_ref[0], p, (((0,), (1,)), ((), ())),
                             preferred_element_type=f32)
    o_ref[0] = ot.T.astype(bf16)


def _post_kernel(ao_ref, res_ref, wo_ref, res2_ref):
    res2_ref[...] = jnp.dot(ao_ref[...], wo_ref[...].astype(bf16),
                            preferred_element_type=f32) + res_ref[...]


def _router_kernel(mx_ref, wr_ref, rw_ref, sel_ref):
    logits = jnp.dot(mx_ref[...], wr_ref[...].astype(bf16),
                     preferred_element_type=f32)
    lane = jax.lax.broadcasted_iota(jnp.int32, (SB, 128), 1)
    logits = jnp.where(lane < E, logits, _NEG)
    lm = jnp.max(logits, axis=1, keepdims=True)
    ex = jnp.exp(logits - lm)
    p = ex / jnp.sum(ex, axis=1, keepdims=True)
    m1 = jnp.max(p, axis=1, keepdims=True)
    a1 = jnp.min(jnp.where(p == m1, lane, 127), axis=1, keepdims=True)
    p2 = jnp.where(lane == a1, -1.0, p)
    m2 = jnp.max(p2, axis=1, keepdims=True)
    a2 = jnp.min(jnp.where(p2 == m2, lane, 127), axis=1, keepdims=True)
    tot = m1 + m2
    rw_ref[...] = jnp.concatenate([m1 / tot, m2 / tot], axis=1)
    sel_ref[...] = jnp.concatenate([a1, a2], axis=1)


def _expert_kernel(be_ref, ids_ref, mx_ref, wg_ref, wu_ref, wd_ref,
                   y_ref, xs_ref, gb_ref, ub_ref, db_ref):
    b = pl.program_id(0)
    changed = jnp.logical_or(b == 0, be_ref[b] != be_ref[jnp.maximum(b - 1, 0)])

    @pl.when(changed)
    def _():
        gb_ref[...] = wg_ref[0].astype(bf16)
        ub_ref[...] = wu_ref[0].astype(bf16)
        db_ref[...] = wd_ref[0].astype(bf16)

    def gather_row(i, _):
        t = ids_ref[b * BLK + i]
        xs_ref[i, :] = mx_ref[t, :]
        return 0

    jax.lax.fori_loop(0, BLK, gather_row, 0)

    xb = xs_ref[...].astype(bf16)
    g = jnp.dot(xb, gb_ref[...], preferred_element_type=f32)
    u = jnp.dot(xb, ub_ref[...], preferred_element_type=f32)
    hdn = (g * jax.nn.sigmoid(g) * u).astype(bf16)
    y_ref[...] = jnp.dot(hdn, db_ref[...], preferred_element_type=f32)


def _combine_kernel(pa_ref, pb_ref, wa_ref, wb_ref, y_ref, o_ref):
    b = pl.program_id(0)

    def row(i, _):
        t = b * SB + i
        wa = jnp.bfloat16(wa_ref[t]).astype(f32)
        wb = jnp.bfloat16(wb_ref[t]).astype(f32)
        ya = y_ref[pa_ref[t], :].astype(bf16).astype(f32)
        yb = y_ref[pb_ref[t], :].astype(bf16).astype(f32)
        o_ref[i, :] = ya * wa + yb * wb
        return 0

    jax.lax.fori_loop(0, SB, row, 0)


def kernel(positions, hidden_states, residual, input_ln_w, post_ln_w, Wq, Wk, Wv, Wo,
           q_norm_w, k_norm_w, Wr, w_gate, w_up, w_down):
    h2 = hidden_states.reshape(S, H)
    r2 = residual.reshape(S, H)

    res = h2 + r2
    xn = _rmsnorm(res, input_ln_w).astype(bf16)

    nsb = S // SB
    qraw, kraw, v = pl.pallas_call(
        _qkv_kernel,
        grid=(nsb,),
        in_specs=[
            pl.BlockSpec((SB, H), lambda i: (i, 0)),
            pl.BlockSpec((H, NH * HD), lambda i: (0, 0)),
            pl.BlockSpec((H, NKV * HD), lambda i: (0, 0)),
            pl.BlockSpec((H, NKV * HD), lambda i: (0, 0)),
        ],
        out_specs=[
            pl.BlockSpec((SB, NH * HD), lambda i: (i, 0)),
            pl.BlockSpec((SB, NKV * HD), lambda i: (i, 0)),
            pl.BlockSpec((SB, NKV * HD), lambda i: (i, 0)),
        ],
        out_shape=[
            jax.ShapeDtypeStruct((S, NH * HD), f32),
            jax.ShapeDtypeStruct((S, NKV * HD), f32),
            jax.ShapeDtypeStruct((S, NKV * HD), bf16),
        ],
    )(xn, Wq, Wk, Wv)

    pos = positions.reshape(1, S)
    qn = _rope(_rmsnorm(qraw.reshape(1, S, NH, HD), q_norm_w), pos)
    kn = _rope(_rmsnorm(kraw.reshape(1, S, NKV, HD), k_norm_w), pos)
    q3 = qn.reshape(S, NH, HD).astype(bf16).transpose(1, 0, 2)
    k3 = kn.reshape(S, NKV, HD).astype(bf16).transpose(1, 0, 2)
    v3 = v.reshape(S, NKV, HD).transpose(1, 0, 2)

    ao = pl.pallas_call(
        _attn_kernel,
        grid=(NH, S // BQ),
        in_specs=[
            pl.BlockSpec((1, BQ, HD), lambda h, i: (h, i, 0)),
            pl.BlockSpec((1, S, HD), lambda h, i: (h // REP, 0, 0)),
            pl.BlockSpec((1, S, HD), lambda h, i: (h // REP, 0, 0)),
        ],
        out_specs=pl.BlockSpec((1, BQ, HD), lambda h, i: (h, i, 0)),
        out_shape=jax.ShapeDtypeStruct((NH, S, HD), bf16),
        scratch_shapes=[pltpu.VMEM((BQ, 1), f32)],
    )(q3, k3, v3)

    aof = ao.transpose(1, 0, 2).reshape(S, NH * HD)

    res2 = pl.pallas_call(
        _post_kernel,
        grid=(nsb,),
        in_specs=[
            pl.BlockSpec((SB, NH * HD), lambda i: (i, 0)),
            pl.BlockSpec((SB, H), lambda i: (i, 0)),
            pl.BlockSpec((NH * HD, H), lambda i: (0, 0)),
        ],
        out_specs=pl.BlockSpec((SB, H), lambda i: (i, 0)),
        out_shape=jax.ShapeDtypeStruct((S, H), f32),
    )(aof, res, Wo)

    mx = _rmsnorm(res2, post_ln_w)

    wr_pad = jnp.zeros((H, 128), f32).at[:, :E].set(Wr)
    rw, sel = pl.pallas_call(
        _router_kernel,
        grid=(nsb,),
        in_specs=[
            pl.BlockSpec((SB, H), lambda i: (i, 0)),
            pl.BlockSpec((H, 128), lambda i: (0, 0)),
        ],
        out_specs=[
            pl.BlockSpec((SB, TOPK), lambda i: (i, 0)),
            pl.BlockSpec((SB, TOPK), lambda i: (i, 0)),
        ],
        out_shape=[
            jax.ShapeDtypeStruct((S, TOPK), f32),
            jax.ShapeDtypeStruct((S, TOPK), jnp.int32),
        ],
    )(mx.astype(bf16), wr_pad)

    e = sel.reshape(-1)
    w = rw.reshape(-1)
    tok = jnp.arange(S * TOPK, dtype=jnp.int32) // TOPK
    onehot = (e[:, None] == jnp.arange(E, dtype=jnp.int32)[None, :]).astype(jnp.int32)
    rank = jnp.sum(onehot * jnp.cumsum(onehot, axis=0), axis=1) - 1
    counts = jnp.sum(onehot, axis=0)
    pc = ((counts + BLK - 1) // BLK) * BLK
    bounds = jnp.cumsum(pc)
    starts = bounds - pc
    posn = starts[e] + rank
    ids = jnp.zeros((PN,), jnp.int32).at[posn].set(tok)
    block_expert = jnp.clip(
        jnp.searchsorted(bounds, jnp.arange(NB, dtype=jnp.int32) * BLK, side='right'),
        0, E - 1).astype(jnp.int32)
    pa = posn[0::TOPK]
    pb = posn[1::TOPK]
    wa = w[0::TOPK]
    wb = w[1::TOPK]

    y = pl.pallas_call(
        _expert_kernel,
        grid_spec=pltpu.PrefetchScalarGridSpec(
            num_scalar_prefetch=2,
            grid=(NB,),
            in_specs=[
                pl.BlockSpec((S, H), lambda b, be, ids: (0, 0)),
                pl.BlockSpec((1, H, I), lambda b, be, ids: (be[b], 0, 0)),
                pl.BlockSpec((1, H, I), lambda b, be, ids: (be[b], 0, 0)),
                pl.BlockSpec((1, I, H), lambda b, be, ids: (be[b], 0, 0)),
            ],
            out_specs=pl.BlockSpec((BLK, H), lambda b, be, ids: (b, 0)),
            scratch_shapes=[
                pltpu.VMEM((BLK, H), f32),
                pltpu.VMEM((H, I), bf16),
                pltpu.VMEM((H, I), bf16),
                pltpu.VMEM((I, H), bf16),
            ],
        ),
        out_shape=jax.ShapeDtypeStruct((PN, H), f32),
    )(block_expert, ids, mx, w_gate, w_up, w_down)

    out = pl.pallas_call(
        _combine_kernel,
        grid_spec=pltpu.PrefetchScalarGridSpec(
            num_scalar_prefetch=4,
            grid=(nsb,),
            in_specs=[pl.BlockSpec((PN, H), lambda b, *_: (0, 0))],
            out_specs=pl.BlockSpec((SB, H), lambda b, *_: (b, 0)),
        ),
        out_shape=jax.ShapeDtypeStruct((S, H), f32),
    )(pa, pb, wa, wb, y)

    return out.reshape(B, S, H), res2.reshape(B, S, H)
